# baseline (speedup 1.0000x reference)
import jax
import jax.numpy as jnp
from jax import lax
from jax.experimental import pallas as pl
from jax.experimental.pallas import tpu as pltpu
from jax.experimental.pallas import tpu_sc as plsc

N_ROWS = 100000
B = 16384
NC, NS, L = 2, 16, 16
NW = NC * NS
BPW = B // NW
NCHUNK = BPW // 128
NGROUP = BPW // L

_A_COEF = (1.0, -1.0 / 6.0, 1.0 / 120.0, -1.0 / 5040.0,
           1.0 / 362880.0, -1.0 / 39916800.0)
_B_COEF = (0.5, -1.0 / 24.0, 1.0 / 720.0, -1.0 / 40320.0,
           1.0 / 3628800.0, -1.0 / 479001600.0)


def _poly(t, coef):
    acc = jnp.full((L,), coef[-1], jnp.float32)
    for c in reversed(coef[:-1]):
        acc = acc * t + c
    return acc


def _body1(idx_hbm, w0_hbm, w1_hbm, w2_hbm, *refs):
    pout = refs[0:9]
    idx_v = refs[9]
    wcol = refs[10:13]
    pcol = refs[13:22]
    sem = refs[22]

    wid = lax.axis_index("s") * NC + lax.axis_index("c")
    base = wid * BPW
    pltpu.sync_copy(idx_hbm.at[pl.ds(wid * NCHUNK, NCHUNK)], idx_v)

    win = (w0_hbm, w1_hbm, w2_hbm)
    copies = []
    for c in range(NCHUNK):
        sl = pl.ds(c * 128, 128)
        ic = idx_v.at[c]
        for d in range(3):
            copies.append(pltpu.async_copy(win[d].at[ic], wcol[d].at[sl], sem))
    for cp in copies:
        cp.wait()

    def group(g, carry):
        sl = pl.ds(g * L, L)
        w0 = wcol[0][sl]
        w1 = wcol[1][sl]
        w2 = wcol[2][sl]
        w00, w11, w22 = w0 * w0, w1 * w1, w2 * w2
        t = w00 + w11 + w22
        A = _poly(t, _A_COEF)
        Bc = _poly(t, _B_COEF)
        w01, w02, w12 = w0 * w1, w0 * w2, w1 * w2
        a0, a1, a2 = A * w0, A * w1, A * w2
        pcol[0][sl] = 1.0 - Bc * (w11 + w22)
        pcol[1][sl] = Bc * w01 - a2
        pcol[2][sl] = Bc * w02 + a1
        pcol[3][sl] = Bc * w01 + a2
        pcol[4][sl] = 1.0 - Bc * (w00 + w22)
        pcol[5][sl] = Bc * w12 - a0
        pcol[6][sl] = Bc * w02 - a1
        pcol[7][sl] = Bc * w12 + a0
        pcol[8][sl] = 1.0 - Bc * (w00 + w11)
        return carry

    lax.fori_loop(0, NGROUP, group, 0, unroll=False)

    for d in range(9):
        pltpu.sync_copy(pcol[d], pout[d].at[pl.ds(base, BPW)])


def _body2(idx_hbm, *refs):
    rin = refs[0:9]
    sin_ = refs[9:11]
    pin = refs[11:20]
    rout = refs[20:29]
    sout = refs[29:31]
    idx_v = refs[31]
    rcol = refs[32:41]
    scol = refs[41:43]
    pcol = refs[43:52]
    ocol = refs[52:61]
    sem = refs[61]

    wid = lax.axis_index("s") * NC + lax.axis_index("c")
    base = wid * BPW
    pltpu.sync_copy(idx_hbm.at[pl.ds(wid * NCHUNK, NCHUNK)], idx_v)

    copies = []
    for d in range(9):
        copies.append(pltpu.async_copy(pin[d].at[pl.ds(base, BPW)], pcol[d], sem))
    for c in range(NCHUNK):
        sl = pl.ds(c * 128, 128)
        ic = idx_v.at[c]
        for d in range(9):
            copies.append(pltpu.async_copy(rin[d].at[ic], rcol[d].at[sl], sem))
        for d in range(2):
            copies.append(pltpu.async_copy(sin_[d].at[ic], scol[d].at[sl], sem))
    for cp in copies:
        cp.wait()

    def group(g, carry):
        sl = pl.ds(g * L, L)
        r = [rcol[d][sl] for d in range(9)]
        p = [pcol[d][sl] for d in range(9)]
        for i in range(3):
            for j in range(3):
                acc = p[i * 3 + 0] * r[0 * 3 + j]
                acc = acc + p[i * 3 + 1] * r[1 * 3 + j]
                acc = acc + p[i * 3 + 2] * r[2 * 3 + j]
                ocol[i * 3 + j][sl] = acc
        return carry

    lax.fori_loop(0, NGROUP, group, 0, unroll=False)

    for d in range(9):
        pltpu.sync_copy(ocol[d], rout[d].at[pl.ds(base, BPW)])
    for d in range(2):
        pltpu.sync_copy(scol[d], sout[d].at[pl.ds(base, BPW)])


@jax.jit
def _run(idx2d, rplanes, wplanes, splanes):
    mesh = plsc.VectorSubcoreMesh(core_axis_name="c", subcore_axis_name="s",
                                  num_cores=NC, num_subcores=NS)
    f1 = pl.kernel(
        _body1,
        out_type=tuple([jax.ShapeDtypeStruct((B,), jnp.float32)] * 9),
        mesh=mesh,
        scratch_types=[pltpu.VMEM((NCHUNK, 128), jnp.int32)]
        + [pltpu.VMEM((BPW,), jnp.float32)] * 12
        + [pltpu.SemaphoreType.DMA],
    )
    pplanes = f1(idx2d, *wplanes)

    f2 = pl.kernel(
        _body2,
        out_type=tuple([jax.ShapeDtypeStruct((B,), jnp.float32)] * 11),
        mesh=mesh,
        scratch_types=[pltpu.VMEM((NCHUNK, 128), jnp.int32)]
        + [pltpu.VMEM((BPW,), jnp.float32)] * 29
        + [pltpu.SemaphoreType.DMA],
    )
    outs = f2(idx2d, *rplanes, *splanes, *pplanes)
    return outs[0:9], outs[9:11]


def kernel(idx, rotations, perturbations_w, shifts):
    idx2d = idx.astype(jnp.int32).reshape(B // 128, 128)
    rplanes = [rotations[:, i, j] for i in range(3) for j in range(3)]
    wplanes = [perturbations_w[:, c] for c in range(3)]
    splanes = [shifts[:, c] for c in range(2)]
    routs, shout = _run(idx2d, rplanes, wplanes, splanes)
    rots = jnp.stack(routs, axis=-1).reshape(B, 3, 3)
    sh = jnp.stack(shout, axis=-1)
    return rots, sh


# speedup vs baseline: 1.1702x; 1.1702x over previous
import jax
import jax.numpy as jnp
from jax import lax
from jax.experimental import pallas as pl
from jax.experimental.pallas import tpu as pltpu
from jax.experimental.pallas import tpu_sc as plsc

N_ROWS = 100000
B = 16384
NC, NS, L = 2, 16, 16
NW = NC * NS
BPW = B // NW
NCHUNK = BPW // 128
NGROUP = BPW // L

_A_COEF = (1.0, -1.0 / 6.0, 1.0 / 120.0, -1.0 / 5040.0,
           1.0 / 362880.0, -1.0 / 39916800.0)
_B_COEF = (0.5, -1.0 / 24.0, 1.0 / 720.0, -1.0 / 40320.0,
           1.0 / 3628800.0, -1.0 / 479001600.0)


def _poly(t, coef):
    acc = jnp.full((L,), coef[-1], jnp.float32)
    for c in reversed(coef[:-1]):
        acc = acc * t + c
    return acc


def _body1(idx_hbm, ws_hbm, *refs):
    pout = refs[0:9]
    sout = refs[9:11]
    idx_v = refs[11]
    wcol = refs[12:15]
    scol = refs[15:17]
    pcol = refs[17:26]
    sem = refs[26]

    wid = lax.axis_index("s") * NC + lax.axis_index("c")
    base = wid * BPW
    pltpu.sync_copy(idx_hbm.at[pl.ds(wid * NCHUNK, NCHUNK)], idx_v)

    copies = []
    for c in range(NCHUNK):
        sl = pl.ds(c * 128, 128)
        ic = idx_v.at[c]
        for d in range(3):
            copies.append(pltpu.async_copy(
                ws_hbm.at[pl.ds(d * N_ROWS, N_ROWS)].at[ic], wcol[d].at[sl], sem))
        for d in range(2):
            copies.append(pltpu.async_copy(
                ws_hbm.at[pl.ds((3 + d) * N_ROWS, N_ROWS)].at[ic], scol[d].at[sl], sem))
    for cp in copies:
        cp.wait()

    def group(g, carry):
        sl = pl.ds(g * L, L)
        w0 = wcol[0][sl]
        w1 = wcol[1][sl]
        w2 = wcol[2][sl]
        w00, w11, w22 = w0 * w0, w1 * w1, w2 * w2
        t = w00 + w11 + w22
        A = _poly(t, _A_COEF)
        Bc = _poly(t, _B_COEF)
        w01, w02, w12 = w0 * w1, w0 * w2, w1 * w2
        a0, a1, a2 = A * w0, A * w1, A * w2
        pcol[0][sl] = 1.0 - Bc * (w11 + w22)
        pcol[1][sl] = Bc * w01 - a2
        pcol[2][sl] = Bc * w02 + a1
        pcol[3][sl] = Bc * w01 + a2
        pcol[4][sl] = 1.0 - Bc * (w00 + w22)
        pcol[5][sl] = Bc * w12 - a0
        pcol[6][sl] = Bc * w02 - a1
        pcol[7][sl] = Bc * w12 + a0
        pcol[8][sl] = 1.0 - Bc * (w00 + w11)
        return carry

    lax.fori_loop(0, NGROUP, group, 0, unroll=False)

    for d in range(9):
        pltpu.sync_copy(pcol[d], pout[d].at[pl.ds(base, BPW)])
    for d in range(2):
        pltpu.sync_copy(scol[d], sout[d].at[pl.ds(base, BPW)])


def _body2(idx_hbm, rot_hbm, *refs):
    pin = refs[0:9]
    rout = refs[9:18]
    idx_v = refs[18]
    rcol = refs[19:28]
    pcol = refs[28:37]
    ocol = refs[37:46]
    sem = refs[46]

    wid = lax.axis_index("s") * NC + lax.axis_index("c")
    base = wid * BPW
    pltpu.sync_copy(idx_hbm.at[pl.ds(wid * NCHUNK, NCHUNK)], idx_v)

    copies = []
    for d in range(9):
        copies.append(pltpu.async_copy(pin[d].at[pl.ds(base, BPW)], pcol[d], sem))
    for c in range(NCHUNK):
        sl = pl.ds(c * 128, 128)
        ic = idx_v.at[c]
        for d in range(9):
            copies.append(pltpu.async_copy(
                rot_hbm.at[pl.ds(d * N_ROWS, N_ROWS)].at[ic], rcol[d].at[sl], sem))
    for cp in copies:
        cp.wait()

    def group(g, carry):
        sl = pl.ds(g * L, L)
        r = [rcol[d][sl] for d in range(9)]
        p = [pcol[d][sl] for d in range(9)]
        for i in range(3):
            for j in range(3):
                acc = p[i * 3 + 0] * r[0 * 3 + j]
                acc = acc + p[i * 3 + 1] * r[1 * 3 + j]
                acc = acc + p[i * 3 + 2] * r[2 * 3 + j]
                ocol[i * 3 + j][sl] = acc
        return carry

    lax.fori_loop(0, NGROUP, group, 0, unroll=False)

    for d in range(9):
        pltpu.sync_copy(ocol[d], rout[d].at[pl.ds(base, BPW)])


@jax.jit
def _run(idx2d, rot_flat, ws_flat):
    mesh = plsc.VectorSubcoreMesh(core_axis_name="c", subcore_axis_name="s",
                                  num_cores=NC, num_subcores=NS)
    f1 = pl.kernel(
        _body1,
        out_type=tuple([jax.ShapeDtypeStruct((B,), jnp.float32)] * 11),
        mesh=mesh,
        scratch_types=[pltpu.VMEM((NCHUNK, 128), jnp.int32)]
        + [pltpu.VMEM((BPW,), jnp.float32)] * 14
        + [pltpu.SemaphoreType.DMA],
    )
    outs1 = f1(idx2d, ws_flat)
    pplanes, shout = outs1[0:9], outs1[9:11]

    f2 = pl.kernel(
        _body2,
        out_type=tuple([jax.ShapeDtypeStruct((B,), jnp.float32)] * 9),
        mesh=mesh,
        scratch_types=[pltpu.VMEM((NCHUNK, 128), jnp.int32)]
        + [pltpu.VMEM((BPW,), jnp.float32)] * 27
        + [pltpu.SemaphoreType.DMA],
    )
    routs = f2(idx2d, rot_flat, *pplanes)
    return routs, shout


def kernel(idx, rotations, perturbations_w, shifts):
    idx2d = idx.astype(jnp.int32).reshape(B // 128, 128)
    rot_flat = jnp.transpose(rotations, (1, 2, 0)).reshape(9 * N_ROWS)
    ws_flat = jnp.concatenate([perturbations_w.T.reshape(3 * N_ROWS),
                               shifts.T.reshape(2 * N_ROWS)])
    routs, shout = _run(idx2d, rot_flat, ws_flat)
    rots = jnp.stack(routs, axis=-1).reshape(B, 3, 3)
    sh = jnp.stack(shout, axis=-1)
    return rots, sh


# speedup vs baseline: 1.2221x; 1.0444x over previous
import jax
import jax.numpy as jnp
from jax import lax
from jax.experimental import pallas as pl
from jax.experimental.pallas import tpu as pltpu
from jax.experimental.pallas import tpu_sc as plsc

N_ROWS = 100000
B = 16384
NC, NS, L = 2, 16, 16
NW = NC * NS
BPW = B // NW
NCHUNK = BPW // 128
NGROUP = BPW // L

_A_COEF = (1.0, -1.0 / 6.0, 1.0 / 120.0, -1.0 / 5040.0,
           1.0 / 362880.0, -1.0 / 39916800.0)
_B_COEF = (0.5, -1.0 / 24.0, 1.0 / 720.0, -1.0 / 40320.0,
           1.0 / 3628800.0, -1.0 / 479001600.0)


def _poly(t, coef):
    acc = jnp.full((L,), coef[-1], jnp.float32)
    for c in reversed(coef[:-1]):
        acc = acc * t + c
    return acc


def _body1(idx_hbm, w_hbm, s_hbm, *refs):
    pout = refs[0:9]
    sout = refs[9:11]
    idx_v = refs[11]
    wcol = refs[12:15]
    scol = refs[15:17]
    pcol = refs[17:26]
    sem = refs[26]

    wid = lax.axis_index("s") * NC + lax.axis_index("c")
    base = wid * BPW
    pltpu.sync_copy(idx_hbm.at[pl.ds(wid * NCHUNK, NCHUNK)], idx_v)

    copies = []
    for c in range(NCHUNK):
        sl = pl.ds(c * 128, 128)
        ic = idx_v.at[c]
        for d in range(3):
            copies.append(pltpu.async_copy(
                w_hbm.at[pl.ds(d * N_ROWS, N_ROWS)].at[ic], wcol[d].at[sl], sem))
        for d in range(2):
            copies.append(pltpu.async_copy(
                s_hbm.at[pl.ds(d * N_ROWS, N_ROWS)].at[ic], scol[d].at[sl], sem))
    for cp in copies:
        cp.wait()

    def group(g, carry):
        sl = pl.ds(g * L, L)
        w0 = wcol[0][sl]
        w1 = wcol[1][sl]
        w2 = wcol[2][sl]
        w00, w11, w22 = w0 * w0, w1 * w1, w2 * w2
        t = w00 + w11 + w22
        A = _poly(t, _A_COEF)
        Bc = _poly(t, _B_COEF)
        w01, w02, w12 = w0 * w1, w0 * w2, w1 * w2
        a0, a1, a2 = A * w0, A * w1, A * w2
        pcol[0][sl] = 1.0 - Bc * (w11 + w22)
        pcol[1][sl] = Bc * w01 - a2
        pcol[2][sl] = Bc * w02 + a1
        pcol[3][sl] = Bc * w01 + a2
        pcol[4][sl] = 1.0 - Bc * (w00 + w22)
        pcol[5][sl] = Bc * w12 - a0
        pcol[6][sl] = Bc * w02 - a1
        pcol[7][sl] = Bc * w12 + a0
        pcol[8][sl] = 1.0 - Bc * (w00 + w11)
        return carry

    lax.fori_loop(0, NGROUP, group, 0, unroll=False)

    for d in range(9):
        pltpu.sync_copy(pcol[d], pout[d].at[pl.ds(base, BPW)])
    for d in range(2):
        pltpu.sync_copy(scol[d], sout[d].at[pl.ds(base, BPW)])


def _body2(idx_hbm, rot_hbm, *refs):
    pin = refs[0:9]
    rout = refs[9:18]
    idx_v = refs[18]
    rcol = refs[19:28]
    pcol = refs[28:37]
    ocol = refs[37:46]
    sem = refs[46]

    wid = lax.axis_index("s") * NC + lax.axis_index("c")
    base = wid * BPW
    pltpu.sync_copy(idx_hbm.at[pl.ds(wid * NCHUNK, NCHUNK)], idx_v)

    copies = []
    for d in range(9):
        copies.append(pltpu.async_copy(pin[d].at[pl.ds(base, BPW)], pcol[d], sem))
    for c in range(NCHUNK):
        sl = pl.ds(c * 128, 128)
        ic = idx_v.at[c]
        for d in range(9):
            copies.append(pltpu.async_copy(
                rot_hbm.at[pl.ds(d * N_ROWS, N_ROWS)].at[ic], rcol[d].at[sl], sem))
    for cp in copies:
        cp.wait()

    def group(g, carry):
        sl = pl.ds(g * L, L)
        r = [rcol[d][sl] for d in range(9)]
        p = [pcol[d][sl] for d in range(9)]
        for i in range(3):
            for j in range(3):
                acc = p[i * 3 + 0] * r[0 * 3 + j]
                acc = acc + p[i * 3 + 1] * r[1 * 3 + j]
                acc = acc + p[i * 3 + 2] * r[2 * 3 + j]
                ocol[i * 3 + j][sl] = acc
        return carry

    lax.fori_loop(0, NGROUP, group, 0, unroll=False)

    for d in range(9):
        pltpu.sync_copy(ocol[d], rout[d].at[pl.ds(base, BPW)])


@jax.jit
def _run(idx2d, rot_flat, w_flat, s_flat):
    mesh = plsc.VectorSubcoreMesh(core_axis_name="c", subcore_axis_name="s",
                                  num_cores=NC, num_subcores=NS)
    f1 = pl.kernel(
        _body1,
        out_type=tuple([jax.ShapeDtypeStruct((B,), jnp.float32)] * 11),
        mesh=mesh,
        scratch_types=[pltpu.VMEM((NCHUNK, 128), jnp.int32)]
        + [pltpu.VMEM((BPW,), jnp.float32)] * 14
        + [pltpu.SemaphoreType.DMA],
    )
    outs1 = f1(idx2d, w_flat, s_flat)
    pplanes, shout = outs1[0:9], outs1[9:11]

    f2 = pl.kernel(
        _body2,
        out_type=tuple([jax.ShapeDtypeStruct((B,), jnp.float32)] * 9),
        mesh=mesh,
        scratch_types=[pltpu.VMEM((NCHUNK, 128), jnp.int32)]
        + [pltpu.VMEM((BPW,), jnp.float32)] * 27
        + [pltpu.SemaphoreType.DMA],
    )
    routs = f2(idx2d, rot_flat, *pplanes)
    return routs, shout


def kernel(idx, rotations, perturbations_w, shifts):
    idx2d = idx.astype(jnp.int32).reshape(B // 128, 128)
    rot_flat = jnp.transpose(rotations, (1, 2, 0)).reshape(9 * N_ROWS)
    w_flat = perturbations_w.T.reshape(3 * N_ROWS)
    s_flat = shifts.T.reshape(2 * N_ROWS)
    routs, shout = _run(idx2d, rot_flat, w_flat, s_flat)
    rots = jnp.stack(routs, axis=-1).reshape(B, 3, 3)
    sh = jnp.stack(shout, axis=-1)
    return rots, sh


# speedup vs baseline: 1.2710x; 1.0400x over previous
import jax
import jax.numpy as jnp
from jax import lax
from jax.experimental import pallas as pl
from jax.experimental.pallas import tpu as pltpu
from jax.experimental.pallas import tpu_sc as plsc

N_ROWS = 100000
B = 16384
NC, NS, L = 2, 16, 16
NW = NC * NS
BPW = B // NW
NCHUNK = BPW // 128
NGROUP = BPW // L

_A_COEF = (1.0, -1.0 / 6.0, 1.0 / 120.0, -1.0 / 5040.0,
           1.0 / 362880.0, -1.0 / 39916800.0)
_B_COEF = (0.5, -1.0 / 24.0, 1.0 / 720.0, -1.0 / 40320.0,
           1.0 / 3628800.0, -1.0 / 479001600.0)


def _poly(t, coef):
    acc = jnp.full((L,), coef[-1], jnp.float32)
    for c in reversed(coef[:-1]):
        acc = acc * t + c
    return acc


def _body1(idx_hbm, w_hbm, *refs):
    pout = refs[0:9]
    idx_v = refs[9]
    wcol = refs[10:13]
    pcol = refs[13:22]
    sem = refs[22]

    wid = lax.axis_index("s") * NC + lax.axis_index("c")
    base = wid * BPW
    pltpu.sync_copy(idx_hbm.at[pl.ds(wid * NCHUNK, NCHUNK)], idx_v)

    copies = []
    for c in range(NCHUNK):
        sl = pl.ds(c * 128, 128)
        ic = idx_v.at[c]
        for d in range(3):
            copies.append(pltpu.async_copy(
                w_hbm.at[pl.ds(d * N_ROWS, N_ROWS)].at[ic], wcol[d].at[sl], sem))
    for cp in copies:
        cp.wait()

    def group(g, carry):
        sl = pl.ds(g * L, L)
        w0 = wcol[0][sl]
        w1 = wcol[1][sl]
        w2 = wcol[2][sl]
        w00, w11, w22 = w0 * w0, w1 * w1, w2 * w2
        t = w00 + w11 + w22
        A = _poly(t, _A_COEF)
        Bc = _poly(t, _B_COEF)
        w01, w02, w12 = w0 * w1, w0 * w2, w1 * w2
        a0, a1, a2 = A * w0, A * w1, A * w2
        pcol[0][sl] = 1.0 - Bc * (w11 + w22)
        pcol[1][sl] = Bc * w01 - a2
        pcol[2][sl] = Bc * w02 + a1
        pcol[3][sl] = Bc * w01 + a2
        pcol[4][sl] = 1.0 - Bc * (w00 + w22)
        pcol[5][sl] = Bc * w12 - a0
        pcol[6][sl] = Bc * w02 - a1
        pcol[7][sl] = Bc * w12 + a0
        pcol[8][sl] = 1.0 - Bc * (w00 + w11)
        return carry

    lax.fori_loop(0, NGROUP, group, 0, unroll=False)

    for d in range(9):
        pltpu.sync_copy(pcol[d], pout[d].at[pl.ds(base, BPW)])


def _body2(idx_hbm, rot_hbm, s_hbm, *refs):
    pin = refs[0:9]
    rout = refs[9:18]
    sout = refs[18:20]
    idx_v = refs[20]
    rcol = refs[21:30]
    pcol = refs[30:39]
    scol = refs[39:41]
    ocol = refs[41:50]
    sem = refs[50]

    wid = lax.axis_index("s") * NC + lax.axis_index("c")
    base = wid * BPW
    pltpu.sync_copy(idx_hbm.at[pl.ds(wid * NCHUNK, NCHUNK)], idx_v)

    copies = []
    for d in range(9):
        copies.append(pltpu.async_copy(pin[d].at[pl.ds(base, BPW)], pcol[d], sem))
    for c in range(NCHUNK):
        sl = pl.ds(c * 128, 128)
        ic = idx_v.at[c]
        for d in range(9):
            copies.append(pltpu.async_copy(
                rot_hbm.at[pl.ds(d * N_ROWS, N_ROWS)].at[ic], rcol[d].at[sl], sem))
        for d in range(2):
            copies.append(pltpu.async_copy(
                s_hbm.at[pl.ds(d * N_ROWS, N_ROWS)].at[ic], scol[d].at[sl], sem))
    for cp in copies:
        cp.wait()

    def group(g, carry):
        sl = pl.ds(g * L, L)
        r = [rcol[d][sl] for d in range(9)]
        p = [pcol[d][sl] for d in range(9)]
        for i in range(3):
            for j in range(3):
                acc = p[i * 3 + 0] * r[0 * 3 + j]
                acc = acc + p[i * 3 + 1] * r[1 * 3 + j]
                acc = acc + p[i * 3 + 2] * r[2 * 3 + j]
                ocol[i * 3 + j][sl] = acc
        return carry

    lax.fori_loop(0, NGROUP, group, 0, unroll=False)

    for d in range(9):
        pltpu.sync_copy(ocol[d], rout[d].at[pl.ds(base, BPW)])
    for d in range(2):
        pltpu.sync_copy(scol[d], sout[d].at[pl.ds(base, BPW)])


@jax.jit
def _run(idx2d, rot_flat, w_flat, s_flat):
    mesh = plsc.VectorSubcoreMesh(core_axis_name="c", subcore_axis_name="s",
                                  num_cores=NC, num_subcores=NS)
    f1 = pl.kernel(
        _body1,
        out_type=tuple([jax.ShapeDtypeStruct((B,), jnp.float32)] * 9),
        mesh=mesh,
        scratch_types=[pltpu.VMEM((NCHUNK, 128), jnp.int32)]
        + [pltpu.VMEM((BPW,), jnp.float32)] * 12
        + [pltpu.SemaphoreType.DMA],
    )
    pplanes = f1(idx2d, w_flat)

    f2 = pl.kernel(
        _body2,
        out_type=tuple([jax.ShapeDtypeStruct((B,), jnp.float32)] * 11),
        mesh=mesh,
        scratch_types=[pltpu.VMEM((NCHUNK, 128), jnp.int32)]
        + [pltpu.VMEM((BPW,), jnp.float32)] * 29
        + [pltpu.SemaphoreType.DMA],
    )
    outs2 = f2(idx2d, rot_flat, s_flat, *pplanes)
    return outs2[0:9], outs2[9:11]


def kernel(idx, rotations, perturbations_w, shifts):
    idx2d = idx.astype(jnp.int32).reshape(B // 128, 128)
    rot_flat = jnp.transpose(rotations, (1, 2, 0)).reshape(9 * N_ROWS)
    w_flat = perturbations_w.T.reshape(3 * N_ROWS)
    s_flat = shifts.T.reshape(2 * N_ROWS)
    routs, shout = _run(idx2d, rot_flat, w_flat, s_flat)
    rots = jnp.stack(routs, axis=-1).reshape(B, 3, 3)
    sh = jnp.stack(shout, axis=-1)
    return rots, sh
